# baseline (speedup 1.0000x reference)
import functools

import jax
import jax.numpy as jnp
from jax import lax
from jax.experimental import pallas as pl
from jax.experimental.pallas import tpu as pltpu
from jax.experimental.pallas import tpu_sc as plsc

N = 10000
E = 320000
NC, NS, NW = 2, 16, 32
BATCH = 128
NB = 80
E_PAD = NW * NB * BATCH
EPW = NB * BATCH
N_PAD = 10240
RPT = N_PAD // NS
NV = EPW // 16
RBLK = 1280


def _sc_mesh():
    return plsc.VectorSubcoreMesh(core_axis_name="c", subcore_axis_name="s")


_VPARAMS = pltpu.CompilerParams(needs_layout_passes=False)


@functools.lru_cache(maxsize=None)
def _make_agg():
    @functools.partial(
        pl.kernel,
        out_type=jax.ShapeDtypeStruct((NC, N_PAD, 128), jnp.float32),
        mesh=_sc_mesh(),
        scratch_types=[
            pltpu.VMEM((EPW,), jnp.int32),
            pltpu.VMEM((BATCH,), jnp.int32),
            pltpu.VMEM((BATCH,), jnp.int32),
            pltpu.VMEM((BATCH,), jnp.int32),
            pltpu.VMEM((BATCH,), jnp.int32),
            pltpu.VMEM((BATCH, 128), jnp.float32),
            pltpu.VMEM((BATCH, 128), jnp.float32),
            pltpu.VMEM_SHARED((N_PAD, 128), jnp.float32),
            pltpu.SemaphoreType.DMA,
            pltpu.SemaphoreType.DMA,
        ],
        compiler_params=_VPARAMS,
    )
    def agg(y_hbm, pk_hbm, zeros_hbm, out_hbm,
            pk_v, su_a, du_a, su_b, du_b, rows_a, rows_b, acc_sh,
            sem_a, sem_b):
        c = lax.axis_index("c")
        s = lax.axis_index("s")
        w = c * NS + s
        pltpu.sync_copy(zeros_hbm.at[pl.ds(s * RPT, RPT)],
                        acc_sh.at[pl.ds(s * RPT, RPT)])
        pltpu.sync_copy(pk_hbm.at[w], pk_v)
        plsc.subcore_barrier()
        mask = jnp.full((16,), 16383, jnp.int32)

        def unpack(j, su, du):
            for k in range(BATCH // 16):
                v = pk_v[pl.ds(j * BATCH + k * 16, 16)]
                su[pl.ds(k * 16, 16)] = lax.bitwise_and(v, mask)
                du[pl.ds(k * 16, 16)] = lax.shift_right_logical(v, 14)

        def gather(su, rows, sem):
            pltpu.make_async_copy(y_hbm.at[su], rows, sem).start()

        def gwait(su, rows, sem):
            pltpu.make_async_copy(y_hbm.at[su], rows, sem).wait()

        unpack(0, su_a, du_a)
        gather(su_a, rows_a, sem_a)

        def body(t, carry):
            j = 2 * t
            unpack(j + 1, su_b, du_b)
            gather(su_b, rows_b, sem_b)
            gwait(su_a, rows_a, sem_a)
            pltpu.sync_copy(rows_a, acc_sh.at[du_a], add=True)

            @pl.when(t < NB // 2 - 1)
            def _():
                unpack(j + 2, su_a, du_a)
                gather(su_a, rows_a, sem_a)

            gwait(su_b, rows_b, sem_b)
            pltpu.sync_copy(rows_b, acc_sh.at[du_b], add=True)
            return carry

        lax.fori_loop(0, NB // 2, body, 0)
        plsc.subcore_barrier()
        pltpu.sync_copy(acc_sh.at[pl.ds(s * RPT, RPT)],
                        out_hbm.at[c, pl.ds(s * RPT, RPT)])

    return agg


@functools.lru_cache(maxsize=None)
def _make_deg():
    @functools.partial(
        pl.kernel,
        out_type=jax.ShapeDtypeStruct((NC, N_PAD), jnp.float32),
        mesh=_sc_mesh(),
        scratch_types=[
            pltpu.VMEM((EPW,), jnp.int32),
            pltpu.VMEM((N_PAD,), jnp.float32),
            pltpu.VMEM((NS, RPT), jnp.float32),
            pltpu.VMEM((RPT,), jnp.float32),
            pltpu.VMEM_SHARED((NS, N_PAD), jnp.float32),
        ],
        compiler_params=_VPARAMS,
    )
    def deg(dst_hbm, out_hbm, dst_v, d_v, part_v, red_v, stage_sh):
        c = lax.axis_index("c")
        s = lax.axis_index("s")
        w = c * NS + s
        pltpu.sync_copy(dst_hbm.at[w], dst_v)
        zero16 = jnp.zeros((16,), jnp.float32)
        one16 = jnp.ones((16,), jnp.float32)

        def zbody(k, carry):
            d_v[pl.ds(k * 16, 16)] = zero16
            return carry

        lax.fori_loop(0, N_PAD // 16, zbody, 0)

        def body(j, carry):
            idx = dst_v[pl.ds(j * 16, 16)]
            plsc.addupdate_scatter(d_v, [idx], one16)
            return carry

        lax.fori_loop(0, NV, body, 0)
        pltpu.sync_copy(d_v, stage_sh.at[s])
        plsc.subcore_barrier()
        pltpu.sync_copy(stage_sh.at[:, pl.ds(s * RPT, RPT)], part_v)

        def rbody(k, carry):
            tot = part_v[0, pl.ds(k * 16, 16)]
            for r in range(1, NS):
                tot = tot + part_v[r, pl.ds(k * 16, 16)]
            red_v[pl.ds(k * 16, 16)] = tot
            return carry

        lax.fori_loop(0, RPT // 16, rbody, 0)
        pltpu.sync_copy(red_v, out_hbm.at[c, pl.ds(s * RPT, RPT)])

    return deg


@functools.lru_cache(maxsize=None)
def _make_cvec():
    @functools.partial(
        pl.kernel,
        out_type=(jax.ShapeDtypeStruct((N_PAD,), jnp.float32),
                  jax.ShapeDtypeStruct((NC, N_PAD), jnp.float32)),
        mesh=_sc_mesh(),
        scratch_types=[
            pltpu.VMEM((EPW,), jnp.int32),
            pltpu.VMEM((EPW,), jnp.int32),
            pltpu.VMEM((NC, RPT), jnp.float32),
            pltpu.VMEM((RPT,), jnp.float32),
            pltpu.VMEM((N_PAD,), jnp.float32),
            pltpu.VMEM((N_PAD,), jnp.float32),
            pltpu.VMEM((NS, RPT), jnp.float32),
            pltpu.VMEM((RPT,), jnp.float32),
            pltpu.VMEM_SHARED((N_PAD,), jnp.float32),
            pltpu.VMEM_SHARED((NS, N_PAD), jnp.float32),
        ],
        compiler_params=_VPARAMS,
    )
    def cvec(degp_hbm, src_hbm, dst_hbm, dinv_hbm, cpart_hbm,
             src_v, dst_v, dp_v, dv_v, dinvf_v, c_v, part_v, red_v,
             dinv_sh, stage_sh):
        c = lax.axis_index("c")
        s = lax.axis_index("s")
        w = c * NS + s
        pltpu.sync_copy(src_hbm.at[w], src_v)
        pltpu.sync_copy(dst_hbm.at[w], dst_v)
        pltpu.sync_copy(degp_hbm.at[:, pl.ds(s * RPT, RPT)], dp_v)
        zero16 = jnp.zeros((16,), jnp.float32)
        lane = lax.iota(jnp.int32, 16)

        def dbody(k, carry):
            d = dp_v[0, pl.ds(k * 16, 16)] + dp_v[1, pl.ds(k * 16, 16)]
            dv = 1.0 / jnp.maximum(d, 1.0)
            row = s * RPT + k * 16 + lane
            dv_v[pl.ds(k * 16, 16)] = jnp.where(row < N, dv, 0.0)
            return carry

        lax.fori_loop(0, RPT // 16, dbody, 0)
        pltpu.sync_copy(dv_v, dinv_sh.at[pl.ds(s * RPT, RPT)])

        @pl.when(c == 0)
        def _():
            pltpu.sync_copy(dv_v, dinv_hbm.at[pl.ds(s * RPT, RPT)])

        def zbody(k, carry):
            c_v[pl.ds(k * 16, 16)] = zero16
            return carry

        lax.fori_loop(0, N_PAD // 16, zbody, 0)
        plsc.subcore_barrier()
        pltpu.sync_copy(dinv_sh, dinvf_v)

        def body(j, carry):
            vals = plsc.load_gather(dinvf_v, [dst_v[pl.ds(j * 16, 16)]])
            plsc.addupdate_scatter(c_v, [src_v[pl.ds(j * 16, 16)]], vals)
            return carry

        lax.fori_loop(0, NV, body, 0)
        pltpu.sync_copy(c_v, stage_sh.at[s])
        plsc.subcore_barrier()
        pltpu.sync_copy(stage_sh.at[:, pl.ds(s * RPT, RPT)], part_v)

        def rbody(k, carry):
            tot = part_v[0, pl.ds(k * 16, 16)]
            for r in range(1, NS):
                tot = tot + part_v[r, pl.ds(k * 16, 16)]
            red_v[pl.ds(k * 16, 16)] = tot
            return carry

        lax.fori_loop(0, RPT // 16, rbody, 0)
        pltpu.sync_copy(red_v, cpart_hbm.at[c, pl.ds(s * RPT, RPT)])

    return cvec


def _mm_body(x_ref, w_ref, o_ref):
    o_ref[...] = jnp.dot(x_ref[...], w_ref[...],
                         preferred_element_type=jnp.float32)


def _mm(x, w):
    return pl.pallas_call(
        _mm_body,
        grid=(N_PAD // RBLK,),
        in_specs=[pl.BlockSpec((RBLK, 128), lambda i: (i, 0)),
                  pl.BlockSpec((128, w.shape[1]), lambda i: (0, 0))],
        out_specs=pl.BlockSpec((RBLK, w.shape[1]), lambda i: (i, 0)),
        out_shape=jax.ShapeDtypeStruct((N_PAD, w.shape[1]), jnp.float32),
    )(x, w)


def _self_body(h_ref, ws_ref, b_ref, o_ref):
    o_ref[...] = jnp.dot(h_ref[...], ws_ref[...],
                         preferred_element_type=jnp.float32) + b_ref[...]


def _self(h, ws, b):
    return pl.pallas_call(
        _self_body,
        grid=(N_PAD // RBLK,),
        in_specs=[pl.BlockSpec((RBLK, 128), lambda i: (i, 0)),
                  pl.BlockSpec((128, 128), lambda i: (0, 0)),
                  pl.BlockSpec((1, 128), lambda i: (0, 0))],
        out_specs=pl.BlockSpec((RBLK, 128), lambda i: (i, 0)),
        out_shape=jax.ShapeDtypeStruct((N_PAD, 128), jnp.float32),
    )(h, ws, b)


def _combine_body(sm_ref, g_ref, dinv_ref, wn_ref, hn_ref):
    aggv = jnp.dot((g_ref[0] + g_ref[1]) * dinv_ref[...], wn_ref[...],
                   preferred_element_type=jnp.float32)
    hn_ref[...] = jnp.maximum(sm_ref[...] + aggv, 0.0)


def _combine(sm, g, dinvb, wn):
    return pl.pallas_call(
        _combine_body,
        grid=(N_PAD // RBLK,),
        in_specs=[pl.BlockSpec((RBLK, 128), lambda i: (i, 0)),
                  pl.BlockSpec((NC, RBLK, 128), lambda i: (0, i, 0)),
                  pl.BlockSpec((RBLK, 128), lambda i: (i, 0)),
                  pl.BlockSpec((128, 128), lambda i: (0, 0))],
        out_specs=pl.BlockSpec((RBLK, 128), lambda i: (i, 0)),
        out_shape=jax.ShapeDtypeStruct((N_PAD, 128), jnp.float32),
    )(sm, g, dinvb, wn)


def _readout_body(sm_ref, g_ref, dinv_ref, wn_ref, c_ref,
                  ws3_ref, wn3_ref, b3_ref, sh_ref, sc_ref, o_ref):
    i = pl.program_id(0)
    aggv = jnp.dot((g_ref[0] + g_ref[1]) * dinv_ref[...], wn_ref[...],
                   preferred_element_type=jnp.float32)
    hn = jnp.maximum(sm_ref[...] + aggv, 0.0)
    row = jax.lax.broadcasted_iota(jnp.int32, (RBLK, 128), 0) + i * RBLK
    hsum = jnp.sum(jnp.where(row < N, hn, 0.0), axis=0, keepdims=True)
    cv = c_ref[0:1, :] + c_ref[1:2, :]
    csum = jnp.dot(cv, hn, preferred_element_type=jnp.float32)

    @pl.when(i == 0)
    def _():
        sh_ref[...] = hsum
        sc_ref[...] = csum

    @pl.when(i > 0)
    def _():
        sh_ref[...] += hsum
        sc_ref[...] += csum

    @pl.when(i == N_PAD // RBLK - 1)
    def _():
        o_ref[...] = (jnp.dot(sh_ref[...] * (1.0 / N), ws3_ref[...],
                              preferred_element_type=jnp.float32)
                      + jnp.dot(sc_ref[...] * (1.0 / N), wn3_ref[...],
                                preferred_element_type=jnp.float32)
                      + b3_ref[...])


def _readout(sm, g, dinvb, wn, cpart, ws3, wn3, b3):
    _, _, out = pl.pallas_call(
        _readout_body,
        grid=(N_PAD // RBLK,),
        in_specs=[pl.BlockSpec((RBLK, 128), lambda i: (i, 0)),
                  pl.BlockSpec((NC, RBLK, 128), lambda i: (0, i, 0)),
                  pl.BlockSpec((RBLK, 128), lambda i: (i, 0)),
                  pl.BlockSpec((128, 128), lambda i: (0, 0)),
                  pl.BlockSpec((NC, RBLK), lambda i: (0, i)),
                  pl.BlockSpec((128, 16), lambda i: (0, 0)),
                  pl.BlockSpec((128, 16), lambda i: (0, 0)),
                  pl.BlockSpec((1, 16), lambda i: (0, 0))],
        out_specs=[pl.BlockSpec((1, 128), lambda i: (0, 0)),
                   pl.BlockSpec((1, 128), lambda i: (0, 0)),
                   pl.BlockSpec((1, 16), lambda i: (0, 0))],
        out_shape=[jax.ShapeDtypeStruct((1, 128), jnp.float32),
                   jax.ShapeDtypeStruct((1, 128), jnp.float32),
                   jax.ShapeDtypeStruct((1, 16), jnp.float32)],
    )(sm, g, dinvb, wn, cpart, ws3, wn3, b3)
    return out


def _pad2(w, r, c):
    return jnp.zeros((r, c), w.dtype).at[:w.shape[0], :w.shape[1]].set(w)


def _pad_row(b, c):
    return jnp.zeros((1, c), b.dtype).at[0, :b.shape[0]].set(b)


def kernel(x, edge_index,
           W_self0, W_neigh0, b0,
           W_self1, W_neigh1, b1,
           W_self2, W_neigh2, b2,
           W_self3, W_neigh3, b3):
    src = edge_index[0].astype(jnp.int32)
    dst = edge_index[1].astype(jnp.int32)
    pad = E_PAD - E
    dump = N + jnp.arange(pad, dtype=jnp.int32) % (N_PAD - N)
    src2 = jnp.concatenate([src, dump]).reshape(NW, EPW)
    dst2 = jnp.concatenate([dst, dump]).reshape(NW, EPW)
    pk2 = src2 | (dst2 << 14)
    zeros128 = jnp.zeros((N_PAD, 128), jnp.float32)

    wsp = [_pad2(W_self0, 128, 128), _pad2(W_self1, 128, 128),
           _pad2(W_self2, 128, 128)]
    wnp = [_pad2(W_neigh0, 128, 128), _pad2(W_neigh1, 128, 128),
           _pad2(W_neigh2, 128, 128)]
    bp = [_pad_row(b0, 128), _pad_row(b1, 128), _pad_row(b2, 128)]
    ws3p = _pad2(W_self3, 128, 16)
    wn3p = _pad2(W_neigh3, 128, 16)
    b3p = _pad_row(b3, 16)

    agg128 = _make_agg()

    degp = _make_deg()(dst2)
    dinv, cpart = _make_cvec()(degp, src2, dst2)
    dinvb = jnp.broadcast_to(dinv[:, None], (N_PAD, 128))
    h = jnp.zeros((N_PAD, 128), jnp.float32).at[:N].set(x)
    for li in range(2):
        g = agg128(h, pk2, zeros128)
        sm = _self(h, wsp[li], bp[li])
        h = _combine(sm, g, dinvb, wnp[li])
    g = agg128(h, pk2, zeros128)
    sm = _self(h, wsp[2], bp[2])
    out = _readout(sm, g, dinvb, wnp[2], cpart, ws3p, wn3p, b3p)
    return out.reshape(16)


# speedup vs baseline: 1.0052x; 1.0052x over previous
import functools

import jax
import jax.numpy as jnp
from jax import lax
from jax.experimental import pallas as pl
from jax.experimental.pallas import tpu as pltpu
from jax.experimental.pallas import tpu_sc as plsc

N = 10000
E = 320000
NC, NS, NW = 2, 16, 32
BATCH = 128
NB = 80
E_PAD = NW * NB * BATCH
EPW = NB * BATCH
N_PAD = 10240
RPT = N_PAD // NS
NV = EPW // 16
RBLK = 1280


def _sc_mesh():
    return plsc.VectorSubcoreMesh(core_axis_name="c", subcore_axis_name="s")


_VPARAMS = pltpu.CompilerParams(needs_layout_passes=False)


@functools.lru_cache(maxsize=None)
def _make_agg():
    @functools.partial(
        pl.kernel,
        out_type=jax.ShapeDtypeStruct((NC, N_PAD, 128), jnp.float32),
        mesh=_sc_mesh(),
        scratch_types=[
            pltpu.VMEM((EPW,), jnp.int32),
            pltpu.VMEM((BATCH,), jnp.int32),
            pltpu.VMEM((BATCH,), jnp.int32),
            pltpu.VMEM((BATCH,), jnp.int32),
            pltpu.VMEM((BATCH,), jnp.int32),
            pltpu.VMEM((BATCH, 128), jnp.float32),
            pltpu.VMEM((BATCH, 128), jnp.float32),
            pltpu.VMEM_SHARED((N_PAD, 128), jnp.float32),
            pltpu.SemaphoreType.DMA,
            pltpu.SemaphoreType.DMA,
        ],
        compiler_params=_VPARAMS,
    )
    def agg(y_hbm, pk_hbm, zeros_hbm, out_hbm,
            pk_v, su_a, du_a, su_b, du_b, rows_a, rows_b, acc_sh,
            sem_a, sem_b):
        c = lax.axis_index("c")
        s = lax.axis_index("s")
        w = c * NS + s
        pltpu.sync_copy(zeros_hbm.at[pl.ds(s * RPT, RPT)],
                        acc_sh.at[pl.ds(s * RPT, RPT)])
        pltpu.sync_copy(pk_hbm.at[w], pk_v)
        plsc.subcore_barrier()
        mask = jnp.full((16,), 16383, jnp.int32)

        def unpack(j, su, du):
            for k in range(BATCH // 16):
                v = pk_v[pl.ds(j * BATCH + k * 16, 16)]
                su[pl.ds(k * 16, 16)] = lax.bitwise_and(v, mask)
                du[pl.ds(k * 16, 16)] = lax.shift_right_logical(v, 14)

        def gather(su, rows, sem):
            pltpu.make_async_copy(y_hbm.at[su], rows, sem).start()

        def gwait(su, rows, sem):
            pltpu.make_async_copy(y_hbm.at[su], rows, sem).wait()

        unpack(0, su_a, du_a)
        gather(su_a, rows_a, sem_a)

        def body(t, carry):
            j = 2 * t
            unpack(j + 1, su_b, du_b)
            gather(su_b, rows_b, sem_b)
            gwait(su_a, rows_a, sem_a)
            pltpu.sync_copy(rows_a, acc_sh.at[du_a], add=True)

            @pl.when(t < NB // 2 - 1)
            def _():
                unpack(j + 2, su_a, du_a)
                gather(su_a, rows_a, sem_a)

            gwait(su_b, rows_b, sem_b)
            pltpu.sync_copy(rows_b, acc_sh.at[du_b], add=True)
            return carry

        lax.fori_loop(0, NB // 2, body, 0)
        plsc.subcore_barrier()
        pltpu.sync_copy(acc_sh.at[pl.ds(s * RPT, RPT)],
                        out_hbm.at[c, pl.ds(s * RPT, RPT)])

    return agg


@functools.lru_cache(maxsize=None)
def _make_deg():
    @functools.partial(
        pl.kernel,
        out_type=jax.ShapeDtypeStruct((NC, N_PAD), jnp.float32),
        mesh=_sc_mesh(),
        scratch_types=[
            pltpu.VMEM((EPW,), jnp.int32),
            pltpu.VMEM((N_PAD,), jnp.float32),
            pltpu.VMEM((NS, RPT), jnp.float32),
            pltpu.VMEM((RPT,), jnp.float32),
            pltpu.VMEM_SHARED((NS, N_PAD), jnp.float32),
        ],
        compiler_params=_VPARAMS,
    )
    def deg(dst_hbm, out_hbm, dst_v, d_v, part_v, red_v, stage_sh):
        c = lax.axis_index("c")
        s = lax.axis_index("s")
        w = c * NS + s
        pltpu.sync_copy(dst_hbm.at[w], dst_v)
        zero16 = jnp.zeros((16,), jnp.float32)
        one16 = jnp.ones((16,), jnp.float32)

        def zbody(k, carry):
            d_v[pl.ds(k * 16, 16)] = zero16
            return carry

        lax.fori_loop(0, N_PAD // 16, zbody, 0)

        def body(j, carry):
            idx = dst_v[pl.ds(j * 16, 16)]
            plsc.addupdate_scatter(d_v, [idx], one16)
            return carry

        lax.fori_loop(0, NV, body, 0)
        pltpu.sync_copy(d_v, stage_sh.at[s])
        plsc.subcore_barrier()
        pltpu.sync_copy(stage_sh.at[:, pl.ds(s * RPT, RPT)], part_v)

        def rbody(k, carry):
            tot = part_v[0, pl.ds(k * 16, 16)]
            for r in range(1, NS):
                tot = tot + part_v[r, pl.ds(k * 16, 16)]
            red_v[pl.ds(k * 16, 16)] = tot
            return carry

        lax.fori_loop(0, RPT // 16, rbody, 0)
        pltpu.sync_copy(red_v, out_hbm.at[c, pl.ds(s * RPT, RPT)])

    return deg


@functools.lru_cache(maxsize=None)
def _make_cvec():
    @functools.partial(
        pl.kernel,
        out_type=(jax.ShapeDtypeStruct((N_PAD,), jnp.float32),
                  jax.ShapeDtypeStruct((NC, N_PAD), jnp.float32)),
        mesh=_sc_mesh(),
        scratch_types=[
            pltpu.VMEM((EPW,), jnp.int32),
            pltpu.VMEM((EPW,), jnp.int32),
            pltpu.VMEM((NC, RPT), jnp.float32),
            pltpu.VMEM((RPT,), jnp.float32),
            pltpu.VMEM((N_PAD,), jnp.float32),
            pltpu.VMEM((N_PAD,), jnp.float32),
            pltpu.VMEM((NS, RPT), jnp.float32),
            pltpu.VMEM((RPT,), jnp.float32),
            pltpu.VMEM_SHARED((N_PAD,), jnp.float32),
            pltpu.VMEM_SHARED((NS, N_PAD), jnp.float32),
        ],
        compiler_params=_VPARAMS,
    )
    def cvec(degp_hbm, src_hbm, dst_hbm, dinv_hbm, cpart_hbm,
             src_v, dst_v, dp_v, dv_v, dinvf_v, c_v, part_v, red_v,
             dinv_sh, stage_sh):
        c = lax.axis_index("c")
        s = lax.axis_index("s")
        w = c * NS + s
        pltpu.sync_copy(src_hbm.at[w], src_v)
        pltpu.sync_copy(dst_hbm.at[w], dst_v)
        pltpu.sync_copy(degp_hbm.at[:, pl.ds(s * RPT, RPT)], dp_v)
        zero16 = jnp.zeros((16,), jnp.float32)
        lane = lax.iota(jnp.int32, 16)

        def dbody(k, carry):
            d = dp_v[0, pl.ds(k * 16, 16)] + dp_v[1, pl.ds(k * 16, 16)]
            dv = 1.0 / jnp.maximum(d, 1.0)
            row = s * RPT + k * 16 + lane
            dv_v[pl.ds(k * 16, 16)] = jnp.where(row < N, dv, 0.0)
            return carry

        lax.fori_loop(0, RPT // 16, dbody, 0)
        pltpu.sync_copy(dv_v, dinv_sh.at[pl.ds(s * RPT, RPT)])

        @pl.when(c == 0)
        def _():
            pltpu.sync_copy(dv_v, dinv_hbm.at[pl.ds(s * RPT, RPT)])

        def zbody(k, carry):
            c_v[pl.ds(k * 16, 16)] = zero16
            return carry

        lax.fori_loop(0, N_PAD // 16, zbody, 0)
        plsc.subcore_barrier()
        pltpu.sync_copy(dinv_sh, dinvf_v)

        def body(j, carry):
            vals = plsc.load_gather(dinvf_v, [dst_v[pl.ds(j * 16, 16)]])
            plsc.addupdate_scatter(c_v, [src_v[pl.ds(j * 16, 16)]], vals)
            return carry

        lax.fori_loop(0, NV, body, 0)
        pltpu.sync_copy(c_v, stage_sh.at[s])
        plsc.subcore_barrier()
        pltpu.sync_copy(stage_sh.at[:, pl.ds(s * RPT, RPT)], part_v)

        def rbody(k, carry):
            tot = part_v[0, pl.ds(k * 16, 16)]
            for r in range(1, NS):
                tot = tot + part_v[r, pl.ds(k * 16, 16)]
            red_v[pl.ds(k * 16, 16)] = tot
            return carry

        lax.fori_loop(0, RPT // 16, rbody, 0)
        pltpu.sync_copy(red_v, cpart_hbm.at[c, pl.ds(s * RPT, RPT)])

    return cvec


def _mm_body(x_ref, w_ref, o_ref):
    o_ref[...] = jnp.dot(x_ref[...], w_ref[...],
                         preferred_element_type=jnp.float32)


def _mm(x, w):
    return pl.pallas_call(
        _mm_body,
        grid=(N_PAD // RBLK,),
        in_specs=[pl.BlockSpec((RBLK, 128), lambda i: (i, 0)),
                  pl.BlockSpec((128, w.shape[1]), lambda i: (0, 0))],
        out_specs=pl.BlockSpec((RBLK, w.shape[1]), lambda i: (i, 0)),
        out_shape=jax.ShapeDtypeStruct((N_PAD, w.shape[1]), jnp.float32),
    )(x, w)


def _combine_body(h_ref, g_ref, dinv_ref, wn_ref, ws_ref, b_ref, hn_ref):
    aggv = jnp.dot((g_ref[0] + g_ref[1]) * dinv_ref[...], wn_ref[...],
                   preferred_element_type=jnp.float32)
    hn_ref[...] = jnp.maximum(
        jnp.dot(h_ref[...], ws_ref[...], preferred_element_type=jnp.float32)
        + aggv + b_ref[...], 0.0)


def _combine(h, g, dinvb, wn, ws, b):
    return pl.pallas_call(
        _combine_body,
        grid=(N_PAD // RBLK,),
        in_specs=[pl.BlockSpec((RBLK, 128), lambda i: (i, 0)),
                  pl.BlockSpec((NC, RBLK, 128), lambda i: (0, i, 0)),
                  pl.BlockSpec((RBLK, 128), lambda i: (i, 0)),
                  pl.BlockSpec((128, 128), lambda i: (0, 0)),
                  pl.BlockSpec((128, 128), lambda i: (0, 0)),
                  pl.BlockSpec((1, 128), lambda i: (0, 0))],
        out_specs=pl.BlockSpec((RBLK, 128), lambda i: (i, 0)),
        out_shape=jax.ShapeDtypeStruct((N_PAD, 128), jnp.float32),
    )(h, g, dinvb, wn, ws, b)


def _readout_body(h_ref, g_ref, dinv_ref, wn_ref, ws_ref, b_ref, c_ref,
                  ws3_ref, wn3_ref, b3_ref, sh_ref, sc_ref, o_ref):
    i = pl.program_id(0)
    aggv = jnp.dot((g_ref[0] + g_ref[1]) * dinv_ref[...], wn_ref[...],
                   preferred_element_type=jnp.float32)
    hn = jnp.maximum(
        jnp.dot(h_ref[...], ws_ref[...], preferred_element_type=jnp.float32)
        + aggv + b_ref[...], 0.0)
    row = jax.lax.broadcasted_iota(jnp.int32, (RBLK, 128), 0) + i * RBLK
    hsum = jnp.sum(jnp.where(row < N, hn, 0.0), axis=0, keepdims=True)
    cv = c_ref[0:1, :] + c_ref[1:2, :]
    csum = jnp.dot(cv, hn, preferred_element_type=jnp.float32)

    @pl.when(i == 0)
    def _():
        sh_ref[...] = hsum
        sc_ref[...] = csum

    @pl.when(i > 0)
    def _():
        sh_ref[...] += hsum
        sc_ref[...] += csum

    @pl.when(i == N_PAD // RBLK - 1)
    def _():
        o_ref[...] = (jnp.dot(sh_ref[...] * (1.0 / N), ws3_ref[...],
                              preferred_element_type=jnp.float32)
                      + jnp.dot(sc_ref[...] * (1.0 / N), wn3_ref[...],
                                preferred_element_type=jnp.float32)
                      + b3_ref[...])


def _readout(h, g, dinvb, wn, ws, b, cpart, ws3, wn3, b3):
    _, _, out = pl.pallas_call(
        _readout_body,
        grid=(N_PAD // RBLK,),
        in_specs=[pl.BlockSpec((RBLK, 128), lambda i: (i, 0)),
                  pl.BlockSpec((NC, RBLK, 128), lambda i: (0, i, 0)),
                  pl.BlockSpec((RBLK, 128), lambda i: (i, 0)),
                  pl.BlockSpec((128, 128), lambda i: (0, 0)),
                  pl.BlockSpec((128, 128), lambda i: (0, 0)),
                  pl.BlockSpec((1, 128), lambda i: (0, 0)),
                  pl.BlockSpec((NC, RBLK), lambda i: (0, i)),
                  pl.BlockSpec((128, 16), lambda i: (0, 0)),
                  pl.BlockSpec((128, 16), lambda i: (0, 0)),
                  pl.BlockSpec((1, 16), lambda i: (0, 0))],
        out_specs=[pl.BlockSpec((1, 128), lambda i: (0, 0)),
                   pl.BlockSpec((1, 128), lambda i: (0, 0)),
                   pl.BlockSpec((1, 16), lambda i: (0, 0))],
        out_shape=[jax.ShapeDtypeStruct((1, 128), jnp.float32),
                   jax.ShapeDtypeStruct((1, 128), jnp.float32),
                   jax.ShapeDtypeStruct((1, 16), jnp.float32)],
    )(h, g, dinvb, wn, ws, b, cpart, ws3, wn3, b3)
    return out


def _pad2(w, r, c):
    return jnp.zeros((r, c), w.dtype).at[:w.shape[0], :w.shape[1]].set(w)


def _pad_row(b, c):
    return jnp.zeros((1, c), b.dtype).at[0, :b.shape[0]].set(b)


def kernel(x, edge_index,
           W_self0, W_neigh0, b0,
           W_self1, W_neigh1, b1,
           W_self2, W_neigh2, b2,
           W_self3, W_neigh3, b3):
    src = edge_index[0].astype(jnp.int32)
    dst = edge_index[1].astype(jnp.int32)
    pad = E_PAD - E
    dump = N + jnp.arange(pad, dtype=jnp.int32) % (N_PAD - N)
    src2 = jnp.concatenate([src, dump]).reshape(NW, EPW)
    dst2 = jnp.concatenate([dst, dump]).reshape(NW, EPW)
    pk2 = src2 | (dst2 << 14)
    zeros128 = jnp.zeros((N_PAD, 128), jnp.float32)

    wsp = [_pad2(W_self0, 128, 128), _pad2(W_self1, 128, 128),
           _pad2(W_self2, 128, 128)]
    wnp = [_pad2(W_neigh0, 128, 128), _pad2(W_neigh1, 128, 128),
           _pad2(W_neigh2, 128, 128)]
    bp = [_pad_row(b0, 128), _pad_row(b1, 128), _pad_row(b2, 128)]
    ws3p = _pad2(W_self3, 128, 16)
    wn3p = _pad2(W_neigh3, 128, 16)
    b3p = _pad_row(b3, 16)

    agg128 = _make_agg()

    degp = _make_deg()(dst2)
    dinv, cpart = _make_cvec()(degp, src2, dst2)
    dinvb = jnp.broadcast_to(dinv[:, None], (N_PAD, 128))
    h = jnp.zeros((N_PAD, 128), jnp.float32).at[:N].set(x)
    for li in range(2):
        g = agg128(h, pk2, zeros128)
        h = _combine(h, g, dinvb, wnp[li], wsp[li], bp[li])
    g = agg128(h, pk2, zeros128)
    out = _readout(h, g, dinvb, wnp[2], wsp[2], bp[2], cpart, ws3p, wn3p, b3p)
    return out.reshape(16)


# speedup vs baseline: 1.0134x; 1.0082x over previous
import functools

import jax
import jax.numpy as jnp
from jax import lax
from jax.experimental import pallas as pl
from jax.experimental.pallas import tpu as pltpu
from jax.experimental.pallas import tpu_sc as plsc

N = 10000
E = 320000
NC, NS, NW = 2, 16, 32
BATCH = 128
NB = 80
E_PAD = NW * NB * BATCH
EPW = NB * BATCH
N_PAD = 10240
RPT = N_PAD // NS
NV = EPW // 16
RBLK = 2560


def _sc_mesh():
    return plsc.VectorSubcoreMesh(core_axis_name="c", subcore_axis_name="s")


_VPARAMS = pltpu.CompilerParams(needs_layout_passes=False)


@functools.lru_cache(maxsize=None)
def _make_agg():
    @functools.partial(
        pl.kernel,
        out_type=jax.ShapeDtypeStruct((NC, N_PAD, 128), jnp.float32),
        mesh=_sc_mesh(),
        scratch_types=[
            pltpu.VMEM((EPW,), jnp.int32),
            pltpu.VMEM((BATCH,), jnp.int32),
            pltpu.VMEM((BATCH,), jnp.int32),
            pltpu.VMEM((BATCH,), jnp.int32),
            pltpu.VMEM((BATCH,), jnp.int32),
            pltpu.VMEM((BATCH, 128), jnp.float32),
            pltpu.VMEM((BATCH, 128), jnp.float32),
            pltpu.VMEM_SHARED((N_PAD, 128), jnp.float32),
            pltpu.SemaphoreType.DMA,
            pltpu.SemaphoreType.DMA,
        ],
        compiler_params=_VPARAMS,
    )
    def agg(y_hbm, pk_hbm, zeros_hbm, out_hbm,
            pk_v, su_a, du_a, su_b, du_b, rows_a, rows_b, acc_sh,
            sem_a, sem_b):
        c = lax.axis_index("c")
        s = lax.axis_index("s")
        w = c * NS + s
        pltpu.sync_copy(zeros_hbm.at[pl.ds(s * RPT, RPT)],
                        acc_sh.at[pl.ds(s * RPT, RPT)])
        pltpu.sync_copy(pk_hbm.at[w], pk_v)
        plsc.subcore_barrier()
        mask = jnp.full((16,), 16383, jnp.int32)

        def unpack(j, su, du):
            for k in range(BATCH // 16):
                v = pk_v[pl.ds(j * BATCH + k * 16, 16)]
                su[pl.ds(k * 16, 16)] = lax.bitwise_and(v, mask)
                du[pl.ds(k * 16, 16)] = lax.shift_right_logical(v, 14)

        def gather(su, rows, sem):
            pltpu.make_async_copy(y_hbm.at[su], rows, sem).start()

        def gwait(su, rows, sem):
            pltpu.make_async_copy(y_hbm.at[su], rows, sem).wait()

        unpack(0, su_a, du_a)
        gather(su_a, rows_a, sem_a)

        def body(t, carry):
            j = 2 * t
            unpack(j + 1, su_b, du_b)
            gather(su_b, rows_b, sem_b)
            gwait(su_a, rows_a, sem_a)
            pltpu.sync_copy(rows_a, acc_sh.at[du_a], add=True)

            @pl.when(t < NB // 2 - 1)
            def _():
                unpack(j + 2, su_a, du_a)
                gather(su_a, rows_a, sem_a)

            gwait(su_b, rows_b, sem_b)
            pltpu.sync_copy(rows_b, acc_sh.at[du_b], add=True)
            return carry

        lax.fori_loop(0, NB // 2, body, 0)
        plsc.subcore_barrier()
        pltpu.sync_copy(acc_sh.at[pl.ds(s * RPT, RPT)],
                        out_hbm.at[c, pl.ds(s * RPT, RPT)])

    return agg


@functools.lru_cache(maxsize=None)
def _make_deg():
    @functools.partial(
        pl.kernel,
        out_type=jax.ShapeDtypeStruct((NC, N_PAD), jnp.float32),
        mesh=_sc_mesh(),
        scratch_types=[
            pltpu.VMEM((EPW,), jnp.int32),
            pltpu.VMEM((N_PAD,), jnp.float32),
            pltpu.VMEM((NS, RPT), jnp.float32),
            pltpu.VMEM((RPT,), jnp.float32),
            pltpu.VMEM_SHARED((NS, N_PAD), jnp.float32),
        ],
        compiler_params=_VPARAMS,
    )
    def deg(dst_hbm, out_hbm, dst_v, d_v, part_v, red_v, stage_sh):
        c = lax.axis_index("c")
        s = lax.axis_index("s")
        w = c * NS + s
        pltpu.sync_copy(dst_hbm.at[w], dst_v)
        zero16 = jnp.zeros((16,), jnp.float32)
        one16 = jnp.ones((16,), jnp.float32)

        def zbody(k, carry):
            d_v[pl.ds(k * 16, 16)] = zero16
            return carry

        lax.fori_loop(0, N_PAD // 16, zbody, 0)

        def body(j, carry):
            idx = dst_v[pl.ds(j * 16, 16)]
            plsc.addupdate_scatter(d_v, [idx], one16)
            return carry

        lax.fori_loop(0, NV, body, 0)
        pltpu.sync_copy(d_v, stage_sh.at[s])
        plsc.subcore_barrier()
        pltpu.sync_copy(stage_sh.at[:, pl.ds(s * RPT, RPT)], part_v)

        def rbody(k, carry):
            tot = part_v[0, pl.ds(k * 16, 16)]
            for r in range(1, NS):
                tot = tot + part_v[r, pl.ds(k * 16, 16)]
            red_v[pl.ds(k * 16, 16)] = tot
            return carry

        lax.fori_loop(0, RPT // 16, rbody, 0)
        pltpu.sync_copy(red_v, out_hbm.at[c, pl.ds(s * RPT, RPT)])

    return deg


@functools.lru_cache(maxsize=None)
def _make_cvec():
    @functools.partial(
        pl.kernel,
        out_type=(jax.ShapeDtypeStruct((N_PAD,), jnp.float32),
                  jax.ShapeDtypeStruct((NC, N_PAD), jnp.float32)),
        mesh=_sc_mesh(),
        scratch_types=[
            pltpu.VMEM((EPW,), jnp.int32),
            pltpu.VMEM((EPW,), jnp.int32),
            pltpu.VMEM((NC, RPT), jnp.float32),
            pltpu.VMEM((RPT,), jnp.float32),
            pltpu.VMEM((N_PAD,), jnp.float32),
            pltpu.VMEM((N_PAD,), jnp.float32),
            pltpu.VMEM((NS, RPT), jnp.float32),
            pltpu.VMEM((RPT,), jnp.float32),
            pltpu.VMEM_SHARED((N_PAD,), jnp.float32),
            pltpu.VMEM_SHARED((NS, N_PAD), jnp.float32),
        ],
        compiler_params=_VPARAMS,
    )
    def cvec(degp_hbm, src_hbm, dst_hbm, dinv_hbm, cpart_hbm,
             src_v, dst_v, dp_v, dv_v, dinvf_v, c_v, part_v, red_v,
             dinv_sh, stage_sh):
        c = lax.axis_index("c")
        s = lax.axis_index("s")
        w = c * NS + s
        pltpu.sync_copy(src_hbm.at[w], src_v)
        pltpu.sync_copy(dst_hbm.at[w], dst_v)
        pltpu.sync_copy(degp_hbm.at[:, pl.ds(s * RPT, RPT)], dp_v)
        zero16 = jnp.zeros((16,), jnp.float32)
        lane = lax.iota(jnp.int32, 16)

        def dbody(k, carry):
            d = dp_v[0, pl.ds(k * 16, 16)] + dp_v[1, pl.ds(k * 16, 16)]
            dv = 1.0 / jnp.maximum(d, 1.0)
            row = s * RPT + k * 16 + lane
            dv_v[pl.ds(k * 16, 16)] = jnp.where(row < N, dv, 0.0)
            return carry

        lax.fori_loop(0, RPT // 16, dbody, 0)
        pltpu.sync_copy(dv_v, dinv_sh.at[pl.ds(s * RPT, RPT)])

        @pl.when(c == 0)
        def _():
            pltpu.sync_copy(dv_v, dinv_hbm.at[pl.ds(s * RPT, RPT)])

        def zbody(k, carry):
            c_v[pl.ds(k * 16, 16)] = zero16
            return carry

        lax.fori_loop(0, N_PAD // 16, zbody, 0)
        plsc.subcore_barrier()
        pltpu.sync_copy(dinv_sh, dinvf_v)

        def body(j, carry):
            vals = plsc.load_gather(dinvf_v, [dst_v[pl.ds(j * 16, 16)]])
            plsc.addupdate_scatter(c_v, [src_v[pl.ds(j * 16, 16)]], vals)
            return carry

        lax.fori_loop(0, NV, body, 0)
        pltpu.sync_copy(c_v, stage_sh.at[s])
        plsc.subcore_barrier()
        pltpu.sync_copy(stage_sh.at[:, pl.ds(s * RPT, RPT)], part_v)

        def rbody(k, carry):
            tot = part_v[0, pl.ds(k * 16, 16)]
            for r in range(1, NS):
                tot = tot + part_v[r, pl.ds(k * 16, 16)]
            red_v[pl.ds(k * 16, 16)] = tot
            return carry

        lax.fori_loop(0, RPT // 16, rbody, 0)
        pltpu.sync_copy(red_v, cpart_hbm.at[c, pl.ds(s * RPT, RPT)])

    return cvec


def _mm_body(x_ref, w_ref, o_ref):
    o_ref[...] = jnp.dot(x_ref[...], w_ref[...],
                         preferred_element_type=jnp.float32)


def _mm(x, w):
    return pl.pallas_call(
        _mm_body,
        grid=(N_PAD // RBLK,),
        in_specs=[pl.BlockSpec((RBLK, 128), lambda i: (i, 0)),
                  pl.BlockSpec((128, w.shape[1]), lambda i: (0, 0))],
        out_specs=pl.BlockSpec((RBLK, w.shape[1]), lambda i: (i, 0)),
        out_shape=jax.ShapeDtypeStruct((N_PAD, w.shape[1]), jnp.float32),
    )(x, w)


def _combine_body(h_ref, g_ref, dinv_ref, wn_ref, ws_ref, b_ref, hn_ref):
    aggv = jnp.dot((g_ref[0] + g_ref[1]) * dinv_ref[...], wn_ref[...],
                   preferred_element_type=jnp.float32)
    hn_ref[...] = jnp.maximum(
        jnp.dot(h_ref[...], ws_ref[...], preferred_element_type=jnp.float32)
        + aggv + b_ref[...], 0.0)


def _combine(h, g, dinvb, wn, ws, b):
    return pl.pallas_call(
        _combine_body,
        grid=(N_PAD // RBLK,),
        in_specs=[pl.BlockSpec((RBLK, 128), lambda i: (i, 0)),
                  pl.BlockSpec((NC, RBLK, 128), lambda i: (0, i, 0)),
                  pl.BlockSpec((RBLK, 128), lambda i: (i, 0)),
                  pl.BlockSpec((128, 128), lambda i: (0, 0)),
                  pl.BlockSpec((128, 128), lambda i: (0, 0)),
                  pl.BlockSpec((1, 128), lambda i: (0, 0))],
        out_specs=pl.BlockSpec((RBLK, 128), lambda i: (i, 0)),
        out_shape=jax.ShapeDtypeStruct((N_PAD, 128), jnp.float32),
    )(h, g, dinvb, wn, ws, b)


def _readout_body(h_ref, g_ref, dinv_ref, wn_ref, ws_ref, b_ref, c_ref,
                  ws3_ref, wn3_ref, b3_ref, sh_ref, sc_ref, o_ref):
    i = pl.program_id(0)
    aggv = jnp.dot((g_ref[0] + g_ref[1]) * dinv_ref[...], wn_ref[...],
                   preferred_element_type=jnp.float32)
    hn = jnp.maximum(
        jnp.dot(h_ref[...], ws_ref[...], preferred_element_type=jnp.float32)
        + aggv + b_ref[...], 0.0)
    row = jax.lax.broadcasted_iota(jnp.int32, (RBLK, 128), 0) + i * RBLK
    hsum = jnp.sum(jnp.where(row < N, hn, 0.0), axis=0, keepdims=True)
    cv = c_ref[0:1, :] + c_ref[1:2, :]
    csum = jnp.dot(cv, hn, preferred_element_type=jnp.float32)

    @pl.when(i == 0)
    def _():
        sh_ref[...] = hsum
        sc_ref[...] = csum

    @pl.when(i > 0)
    def _():
        sh_ref[...] += hsum
        sc_ref[...] += csum

    @pl.when(i == N_PAD // RBLK - 1)
    def _():
        o_ref[...] = (jnp.dot(sh_ref[...] * (1.0 / N), ws3_ref[...],
                              preferred_element_type=jnp.float32)
                      + jnp.dot(sc_ref[...] * (1.0 / N), wn3_ref[...],
                                preferred_element_type=jnp.float32)
                      + b3_ref[...])


def _readout(h, g, dinvb, wn, ws, b, cpart, ws3, wn3, b3):
    _, _, out = pl.pallas_call(
        _readout_body,
        grid=(N_PAD // RBLK,),
        in_specs=[pl.BlockSpec((RBLK, 128), lambda i: (i, 0)),
                  pl.BlockSpec((NC, RBLK, 128), lambda i: (0, i, 0)),
                  pl.BlockSpec((RBLK, 128), lambda i: (i, 0)),
                  pl.BlockSpec((128, 128), lambda i: (0, 0)),
                  pl.BlockSpec((128, 128), lambda i: (0, 0)),
                  pl.BlockSpec((1, 128), lambda i: (0, 0)),
                  pl.BlockSpec((NC, RBLK), lambda i: (0, i)),
                  pl.BlockSpec((128, 16), lambda i: (0, 0)),
                  pl.BlockSpec((128, 16), lambda i: (0, 0)),
                  pl.BlockSpec((1, 16), lambda i: (0, 0))],
        out_specs=[pl.BlockSpec((1, 128), lambda i: (0, 0)),
                   pl.BlockSpec((1, 128), lambda i: (0, 0)),
                   pl.BlockSpec((1, 16), lambda i: (0, 0))],
        out_shape=[jax.ShapeDtypeStruct((1, 128), jnp.float32),
                   jax.ShapeDtypeStruct((1, 128), jnp.float32),
                   jax.ShapeDtypeStruct((1, 16), jnp.float32)],
    )(h, g, dinvb, wn, ws, b, cpart, ws3, wn3, b3)
    return out


def _pad2(w, r, c):
    return jnp.zeros((r, c), w.dtype).at[:w.shape[0], :w.shape[1]].set(w)


def _pad_row(b, c):
    return jnp.zeros((1, c), b.dtype).at[0, :b.shape[0]].set(b)


def kernel(x, edge_index,
           W_self0, W_neigh0, b0,
           W_self1, W_neigh1, b1,
           W_self2, W_neigh2, b2,
           W_self3, W_neigh3, b3):
    src = edge_index[0].astype(jnp.int32)
    dst = edge_index[1].astype(jnp.int32)
    pad = E_PAD - E
    dump = N + jnp.arange(pad, dtype=jnp.int32) % (N_PAD - N)
    src2 = jnp.concatenate([src, dump]).reshape(NW, EPW)
    dst2 = jnp.concatenate([dst, dump]).reshape(NW, EPW)
    pk2 = src2 | (dst2 << 14)
    zeros128 = jnp.zeros((N_PAD, 128), jnp.float32)

    wsp = [_pad2(W_self0, 128, 128), _pad2(W_self1, 128, 128),
           _pad2(W_self2, 128, 128)]
    wnp = [_pad2(W_neigh0, 128, 128), _pad2(W_neigh1, 128, 128),
           _pad2(W_neigh2, 128, 128)]
    bp = [_pad_row(b0, 128), _pad_row(b1, 128), _pad_row(b2, 128)]
    ws3p = _pad2(W_self3, 128, 16)
    wn3p = _pad2(W_neigh3, 128, 16)
    b3p = _pad_row(b3, 16)

    agg128 = _make_agg()

    degp = _make_deg()(dst2)
    dinv, cpart = _make_cvec()(degp, src2, dst2)
    dinvb = jnp.broadcast_to(dinv[:, None], (N_PAD, 128))
    h = jnp.zeros((N_PAD, 128), jnp.float32).at[:N].set(x)
    for li in range(2):
        g = agg128(h, pk2, zeros128)
        h = _combine(h, g, dinvb, wnp[li], wsp[li], bp[li])
    g = agg128(h, pk2, zeros128)
    out = _readout(h, g, dinvb, wnp[2], wsp[2], bp[2], cpart, ws3p, wn3p, b3p)
    return out.reshape(16)
